# Initial kernel scaffold; baseline (speedup 1.0000x reference)
import jax
import jax.numpy as jnp
from jax.experimental import pallas as pl


def kernel(xyz, feature, w1, b1, w2, b2, w3, b3, wl, bl, wf, bf):
    raise NotImplementedError("write your pallas kernel here")


# speedup vs baseline: 4.2784x; 4.2784x over previous
import functools
import numpy as np
import jax
import jax.numpy as jnp
from jax import lax
from jax.experimental import pallas as pl
from jax.experimental.pallas import tpu as pltpu
from jax.experimental.pallas import tpu_sc as plsc

_S = 512
_K = 32
_R2 = np.float32(0.4 ** 2)
_SBLK = 8
_STILE = 128
_NBLK = 256


def _fps_body(xyzT_ref, nxyz_ref, dist_ref, ist_ref):
    B = xyzT_ref.shape[0]
    N = xyzT_ref.shape[2]
    xc = [xyzT_ref[:, c, :] for c in range(3)]
    iota = lax.broadcasted_iota(jnp.int32, (B, N), 1).astype(jnp.float32)
    dist_ref[...] = jnp.full((B, N), 1e10, jnp.float32)
    ist_ref[...] = jnp.zeros((B, 1), jnp.float32)

    def body(i, carry):
        ist = ist_ref[...]
        oh = (iota == ist).astype(jnp.float32)
        cent = [jnp.sum(oh * xc[c], axis=1, keepdims=True) for c in range(3)]
        nxyz_ref[:, pl.ds(i, 1), :] = jnp.concatenate(cent, axis=1)[:, None, :]
        d = ((xc[0] - cent[0]) ** 2 + (xc[1] - cent[1]) ** 2
             + (xc[2] - cent[2]) ** 2)
        dist = jnp.minimum(dist_ref[...], d)
        dist_ref[...] = dist
        m = jnp.max(dist, axis=1, keepdims=True)
        ist_ref[...] = jnp.min(
            jnp.where(dist == m, iota, float(N)), axis=1, keepdims=True)
        return carry

    lax.fori_loop(0, _S, body, 0)


def _run_fps(xyzT):
    B, _, N = xyzT.shape
    return pl.pallas_call(
        _fps_body,
        out_shape=jax.ShapeDtypeStruct((B, _S, 3), jnp.float32),
        scratch_shapes=[pltpu.VMEM((B, N), jnp.float32),
                        pltpu.VMEM((B, 1), jnp.float32)],
    )(xyzT)


def _bq_body(xyzT_ref, xyz_ref, nxyz_ref, w1t_ref, b1_ref, w2t_ref, b2_ref,
             w3t_ref, b3_ref, wts_ref, idx_ref, sel_ref):
    b = pl.program_id(0)
    N = xyzT_ref.shape[2]
    Nf = float(N)
    iota = lax.broadcasted_iota(jnp.int32, (_SBLK, N), 1).astype(jnp.float32)

    d2 = jnp.zeros((_SBLK, N), jnp.float32)
    for c in range(3):
        d2 = d2 + (nxyz_ref[0, :, c:c + 1] - xyzT_ref[0, c:c + 1, :]) ** 2

    running = d2 <= _R2
    idx0 = None
    cols = []
    for k in range(_K):
        m = jnp.min(jnp.where(running, iota, Nf), axis=1, keepdims=True)
        if k == 0:
            idx0 = m
        meff = jnp.where(m >= Nf, idx0, m)
        selk = iota == meff
        sel_ref[k] = selk.astype(jnp.float32)
        running = jnp.logical_and(running, jnp.logical_not(selk))
        cols.append(meff)

    idx = jnp.concatenate(cols, axis=1)
    idx_ref[0] = idx.astype(jnp.int32) + b * N

    xyz = xyz_ref[0]
    grels = []
    for s in range(_SBLK):
        g = jnp.dot(sel_ref[:, s, :], xyz,
                    preferred_element_type=jnp.float32)
        grels.append(g - nxyz_ref[0, s:s + 1, :])
    grel = jnp.concatenate(grels, axis=0)

    h = jnp.maximum(jnp.dot(grel, w1t_ref[...],
                            preferred_element_type=jnp.float32)
                    + b1_ref[...], 0.0)
    h = jnp.maximum(jnp.dot(h, w2t_ref[...],
                            preferred_element_type=jnp.float32)
                    + b2_ref[...], 0.0)
    wt = jnp.dot(h, w3t_ref[...],
                 preferred_element_type=jnp.float32) + b3_ref[...]
    wts_ref[0] = wt.reshape(_SBLK, _K, 64)


def _run_bq(xyzT, xyz, nxyz, w1t, b1r, w2t, b2r, w3t, b3r):
    B, N, _ = xyz.shape
    full = lambda *shape: shape
    return pl.pallas_call(
        _bq_body,
        grid=(B, _S // _SBLK),
        in_specs=[
            pl.BlockSpec((1, 3, N), lambda b, j: (b, 0, 0)),
            pl.BlockSpec((1, N, 3), lambda b, j: (b, 0, 0)),
            pl.BlockSpec((1, _SBLK, 3), lambda b, j: (b, j, 0)),
            pl.BlockSpec((3, 64), lambda b, j: (0, 0)),
            pl.BlockSpec((1, 64), lambda b, j: (0, 0)),
            pl.BlockSpec((64, 64), lambda b, j: (0, 0)),
            pl.BlockSpec((1, 64), lambda b, j: (0, 0)),
            pl.BlockSpec((64, 64), lambda b, j: (0, 0)),
            pl.BlockSpec((1, 64), lambda b, j: (0, 0)),
        ],
        out_specs=[
            pl.BlockSpec((1, _SBLK, _K, 64), lambda b, j: (b, j, 0, 0)),
            pl.BlockSpec((1, _SBLK, _K), lambda b, j: (b, j, 0)),
        ],
        out_shape=[jax.ShapeDtypeStruct((B, _S, _K, 64), jnp.float32),
                   jax.ShapeDtypeStruct((B, _S, _K), jnp.int32)],
        scratch_shapes=[pltpu.VMEM((_K, _SBLK, N), jnp.float32)],
    )(xyzT, xyz, nxyz, w1t, b1r, w2t, b2r, w3t, b3r)


def _gather_rows(table, idx):
    V, D = table.shape
    Btot = idx.shape[0]
    info = plsc.get_sparse_core_info()
    NW = info.num_cores * info.num_subcores
    b_per_w = Btot // NW
    CH = min(256, b_per_w)
    nch = b_per_w // CH
    mesh = plsc.VectorSubcoreMesh(core_axis_name="c", subcore_axis_name="s")

    @functools.partial(
        pl.kernel, mesh=mesh,
        out_type=jax.ShapeDtypeStruct((Btot, D), jnp.float32),
        scratch_types=[pltpu.VMEM((CH,), jnp.int32),
                       pltpu.VMEM((CH, D), jnp.float32),
                       pltpu.SemaphoreType.DMA],
    )
    def gk(table_hbm, idx_hbm, out_hbm, idx_v, rows_v, sem):
        wid = lax.axis_index("s") * info.num_cores + lax.axis_index("c")
        base = wid * b_per_w
        for c in range(nch):
            off = base + c * CH
            pltpu.sync_copy(idx_hbm.at[pl.ds(off, CH)], idx_v)
            pltpu.async_copy(table_hbm.at[idx_v], rows_v, sem).wait()
            pltpu.sync_copy(rows_v, out_hbm.at[pl.ds(off, CH)])

    return gk(table, idx)


def _nf_body(g_ref, wt_ref, wl3_ref, bl_ref, out_ref, nf3_ref, acc_ref):
    def stage_a(s, carry):
        nf3_ref[:, s, :] = lax.dot_general(
            wt_ref[0, s], g_ref[0, s],
            dimension_numbers=(((0,), (0,)), ((), ())),
            preferred_element_type=jnp.float32)
        return carry

    lax.fori_loop(0, _STILE, stage_a, 0)

    acc_ref[...] = jnp.zeros_like(acc_ref)

    def stage_b(j, carry):
        acc_ref[...] += jnp.dot(nf3_ref[j], wl3_ref[j],
                                preferred_element_type=jnp.float32)
        return carry

    lax.fori_loop(0, 64, stage_b, 0)
    out_ref[0] = jnp.maximum(acc_ref[...] + bl_ref[...], 0.0)


def _run_nf(g4, wts, wl3, blr):
    B = wts.shape[0]
    D = g4.shape[3]
    OC = wl3.shape[2]
    return pl.pallas_call(
        _nf_body,
        grid=(B, _S // _STILE),
        in_specs=[
            pl.BlockSpec((1, _STILE, _K, D), lambda b, t: (b, t, 0, 0)),
            pl.BlockSpec((1, _STILE, _K, 64), lambda b, t: (b, t, 0, 0)),
            pl.BlockSpec((64, D, OC), lambda b, t: (0, 0, 0)),
            pl.BlockSpec((1, OC), lambda b, t: (0, 0)),
        ],
        out_specs=pl.BlockSpec((1, _STILE, OC), lambda b, t: (b, t, 0)),
        out_shape=jax.ShapeDtypeStruct((B, _S, OC), jnp.float32),
        scratch_shapes=[pltpu.VMEM((64, _STILE, D), jnp.float32),
                        pltpu.VMEM((_STILE, OC), jnp.float32)],
    )(g4, wts, wl3, blr)


def _interp_body(xyz_ref, nxyzT_ref, nfeat_ref, featT_ref, wfit_ref,
                 wfft_ref, bf_ref, out_ref):
    Sf = float(_S)
    iota = lax.broadcasted_iota(jnp.int32, (_NBLK, _S), 1).astype(jnp.float32)

    d2 = jnp.zeros((_NBLK, _S), jnp.float32)
    for c in range(3):
        d2 = d2 + (xyz_ref[0, :, c:c + 1] - nxyzT_ref[0, c:c + 1, :]) ** 2

    wmat = jnp.zeros((_NBLK, _S), jnp.float32)
    rds = []
    ohs = []
    for j in range(3):
        m = jnp.min(d2, axis=1, keepdims=True)
        ij = jnp.min(jnp.where(d2 == m, iota, Sf), axis=1, keepdims=True)
        oh = iota == ij
        rds.append(1.0 / (m + 1e-8))
        ohs.append(oh)
        d2 = jnp.where(oh, jnp.inf, d2)
    norm = rds[0] + rds[1] + rds[2]
    for j in range(3):
        wmat = wmat + (rds[j] / norm) * ohs[j].astype(jnp.float32)

    interp = jnp.dot(wmat, nfeat_ref[0],
                     preferred_element_type=jnp.float32)
    out = (interp @ wfit_ref[...]
           + jnp.dot(featT_ref[0], wfft_ref[...],
                     preferred_element_type=jnp.float32)
           + bf_ref[...])
    out_ref[0] = jnp.maximum(out, 0.0)


def _run_interp(xyz, nxyzT, nfeat, featT, wfit, wfft, bfr):
    B, N, _ = xyz.shape
    D = featT.shape[2]
    OC = nfeat.shape[2]
    return pl.pallas_call(
        _interp_body,
        grid=(B, N // _NBLK),
        in_specs=[
            pl.BlockSpec((1, _NBLK, 3), lambda b, j: (b, j, 0)),
            pl.BlockSpec((1, 3, _S), lambda b, j: (b, 0, 0)),
            pl.BlockSpec((1, _S, OC), lambda b, j: (b, 0, 0)),
            pl.BlockSpec((1, _NBLK, D), lambda b, j: (b, j, 0)),
            pl.BlockSpec((OC, OC), lambda b, j: (0, 0)),
            pl.BlockSpec((D, OC), lambda b, j: (0, 0)),
            pl.BlockSpec((1, OC), lambda b, j: (0, 0)),
        ],
        out_specs=pl.BlockSpec((1, _NBLK, OC), lambda b, j: (b, j, 0)),
        out_shape=jax.ShapeDtypeStruct((B, N, OC), jnp.float32),
    )(xyz, nxyzT, nfeat, featT, wfit, wfft, bfr)


def kernel(xyz, feature, w1, b1, w2, b2, w3, b3, wl, bl, wf, bf):
    B, N, _ = xyz.shape
    D = feature.shape[1]
    OC = wl.shape[0]

    xyzT = jnp.transpose(xyz, (0, 2, 1))
    featT = jnp.transpose(feature, (0, 2, 1))

    nxyz = _run_fps(xyzT)
    nxyzT = jnp.transpose(nxyz, (0, 2, 1))

    wts, idx = _run_bq(xyzT, xyz, nxyz,
                       w1.T, b1[None, :], w2.T, b2[None, :],
                       w3.T, b3[None, :])

    g = _gather_rows(featT.reshape(B * N, D), idx.reshape(-1))
    g4 = g.reshape(B, _S, _K, D)

    wl3 = jnp.transpose(wl.reshape(OC, D, 64), (2, 1, 0))
    nfeat = _run_nf(g4, wts, wl3, bl[None, :])

    outT = _run_interp(xyz, nxyzT, nfeat, featT,
                       wf[:, :OC].T, wf[:, OC:].T, bf[None, :])
    return jnp.transpose(outT, (0, 2, 1))
